# Initial kernel scaffold; baseline (speedup 1.0000x reference)
import jax
import jax.numpy as jnp
from jax.experimental import pallas as pl


def kernel(x, neighbor, W1x, W1n, W2x, W2n, g1, b1, g2, b2, Wc, bc):
    raise NotImplementedError("write your pallas kernel here")


# speedup vs baseline: 1.8010x; 1.8010x over previous
import functools

import jax
import jax.numpy as jnp
from jax.experimental import pallas as pl
from jax.experimental.pallas import tpu as pltpu

N = 10000
DEG = 16
F = 256
H = 128
C = 40
B = 400
NB = N // B
EPS = 1e-5


def _k1(x_ref, nb_ref, w1xt_ref, w1nt_ref, g1_ref, b1_ref,
        x1_ref, f2_ref, s1_ref, ss1_ref):
    g1 = g1_ref[0, 0]
    b1 = b1_ref[0, 0]
    nb2d = nb_ref[...]
    w1xt = w1xt_ref[...].astype(jnp.bfloat16)
    nb1 = jnp.dot(nb2d.astype(jnp.bfloat16), w1xt,
                  preferred_element_type=jnp.float32)
    nb3 = nb1.reshape(B, DEG, H)
    m = jnp.mean(nb3, axis=(1, 2), keepdims=True)
    d = nb3 - m
    v = jnp.mean(d * d, axis=(1, 2), keepdims=True)
    y = jnp.maximum(d * jax.lax.rsqrt(v + EPS) * g1 + b1, 0.0)
    f2_ref[...] = jnp.mean(y, axis=1)
    f = jnp.mean(nb2d.reshape(B, DEG, F), axis=1)
    x1 = (jnp.dot(x_ref[...].astype(jnp.bfloat16), w1xt,
                  preferred_element_type=jnp.float32)
          + jnp.dot(f.astype(jnp.bfloat16), w1nt_ref[...].astype(jnp.bfloat16),
                    preferred_element_type=jnp.float32))
    x1_ref[...] = x1
    s1_ref[...] = jnp.sum(x1.reshape(B // 8, 8, H), axis=0)
    ss1_ref[...] = jnp.sum((x1 * x1).reshape(B // 8, 8, H), axis=0)


def _k2(x1_ref, f2_ref, s1_ref, ss1_ref, w2xt_ref, w2nt_ref, g1_ref, b1_ref,
        x2_ref, s2_ref, ss2_ref):
    cnt = float(N * H)
    m1 = jnp.sum(s1_ref[...]) / cnt
    v1 = jnp.sum(ss1_ref[...]) / cnt - m1 * m1
    g1 = g1_ref[0, 0]
    b1 = b1_ref[0, 0]
    x1 = x1_ref[...]
    x1n = jnp.maximum((x1 - m1) * jax.lax.rsqrt(v1 + EPS) * g1 + b1, 0.0)
    x2 = (jnp.dot(x1n, w2xt_ref[...], preferred_element_type=jnp.float32)
          + jnp.dot(f2_ref[...], w2nt_ref[...],
                    preferred_element_type=jnp.float32))
    x2_ref[...] = x2
    s2_ref[...] = jnp.sum(x2.reshape(B // 8, 8, H), axis=0)
    ss2_ref[...] = jnp.sum((x2 * x2).reshape(B // 8, 8, H), axis=0)


def _k3(x2_ref, s2_ref, ss2_ref, wct_ref, bc_ref, g2_ref, b2_ref, out_ref):
    cnt = float(N * H)
    m2 = jnp.sum(s2_ref[...]) / cnt
    v2 = jnp.sum(ss2_ref[...]) / cnt - m2 * m2
    g2 = g2_ref[0, 0]
    b2 = b2_ref[0, 0]
    x2 = x2_ref[...]
    x2n = jnp.maximum((x2 - m2) * jax.lax.rsqrt(v2 + EPS) * g2 + b2, 0.0)
    out_ref[...] = (jnp.dot(x2n, wct_ref[...], preferred_element_type=jnp.float32)
                    + bc_ref[...])


def _smem11():
    return pl.BlockSpec(memory_space=pltpu.SMEM)


def _full():
    return pl.BlockSpec(memory_space=pltpu.VMEM)


@functools.partial(jax.jit)
def kernel(x, neighbor, W1x, W1n, W2x, W2n, g1, b1, g2, b2, Wc, bc):
    x2d = x.reshape(N, F)
    nb2d = neighbor.reshape(N * DEG, F)
    g1s = g1.reshape(1, 1)
    b1s = b1.reshape(1, 1)
    g2s = g2.reshape(1, 1)
    b2s = b2.reshape(1, 1)

    x1, f2, s1, ss1 = pl.pallas_call(
        _k1,
        grid=(NB,),
        in_specs=[
            pl.BlockSpec((B, F), lambda i: (i, 0)),
            pl.BlockSpec((B * DEG, F), lambda i: (i, 0)),
            _full(),
            _full(),
            _smem11(),
            _smem11(),
        ],
        out_specs=[
            pl.BlockSpec((B, H), lambda i: (i, 0)),
            pl.BlockSpec((B, H), lambda i: (i, 0)),
            pl.BlockSpec((8, H), lambda i: (i, 0)),
            pl.BlockSpec((8, H), lambda i: (i, 0)),
        ],
        out_shape=[
            jax.ShapeDtypeStruct((N, H), jnp.float32),
            jax.ShapeDtypeStruct((N, H), jnp.float32),
            jax.ShapeDtypeStruct((NB * 8, H), jnp.float32),
            jax.ShapeDtypeStruct((NB * 8, H), jnp.float32),
        ],
        compiler_params=pltpu.CompilerParams(
            dimension_semantics=("arbitrary",)),
    )(x2d, nb2d, W1x.T, W1n.T, g1s, b1s)

    x2, s2, ss2 = pl.pallas_call(
        _k2,
        grid=(NB,),
        in_specs=[
            pl.BlockSpec((B, H), lambda i: (i, 0)),
            pl.BlockSpec((B, H), lambda i: (i, 0)),
            pl.BlockSpec((NB * 8, H), lambda i: (0, 0)),
            pl.BlockSpec((NB * 8, H), lambda i: (0, 0)),
            _full(),
            _full(),
            _smem11(),
            _smem11(),
        ],
        out_specs=[
            pl.BlockSpec((B, H), lambda i: (i, 0)),
            pl.BlockSpec((8, H), lambda i: (i, 0)),
            pl.BlockSpec((8, H), lambda i: (i, 0)),
        ],
        out_shape=[
            jax.ShapeDtypeStruct((N, H), jnp.float32),
            jax.ShapeDtypeStruct((NB * 8, H), jnp.float32),
            jax.ShapeDtypeStruct((NB * 8, H), jnp.float32),
        ],
        compiler_params=pltpu.CompilerParams(
            dimension_semantics=("arbitrary",)),
    )(x1, f2, s1, ss1, W2x.T, W2n.T, g1s, b1s)

    out = pl.pallas_call(
        _k3,
        grid=(NB,),
        in_specs=[
            pl.BlockSpec((B, H), lambda i: (i, 0)),
            pl.BlockSpec((NB * 8, H), lambda i: (0, 0)),
            pl.BlockSpec((NB * 8, H), lambda i: (0, 0)),
            _full(),
            _full(),
            _smem11(),
            _smem11(),
        ],
        out_specs=pl.BlockSpec((B, C), lambda i: (i, 0)),
        out_shape=jax.ShapeDtypeStruct((N, C), jnp.float32),
        compiler_params=pltpu.CompilerParams(
            dimension_semantics=("arbitrary",)),
    )(x2, s2, ss2, Wc.T, bc.reshape(1, C), g2s, b2s)

    return out


# speedup vs baseline: 1.8023x; 1.0007x over previous
import functools

import jax
import jax.numpy as jnp
from jax.experimental import pallas as pl
from jax.experimental.pallas import tpu as pltpu

N = 10000
DEG = 16
F = 256
H = 128
C = 40
B = 400
NB = N // B
EPS = 1e-5


def _k1(x_ref, nb_ref, w1xt_ref, w1nt_ref, g1_ref, b1_ref,
        x1_ref, f2_ref, s1_ref, ss1_ref):
    g1 = g1_ref[0, 0]
    b1 = b1_ref[0, 0]
    nb2d = nb_ref[...]
    w1xt = w1xt_ref[...].astype(jnp.bfloat16)
    nb1 = jnp.dot(nb2d.astype(jnp.bfloat16), w1xt,
                  preferred_element_type=jnp.float32)
    nb3 = nb1.reshape(B, DEG, H)
    m = jnp.mean(nb3, axis=(1, 2), keepdims=True)
    d = nb3 - m
    v = jnp.mean(d * d, axis=(1, 2), keepdims=True)
    y = jnp.maximum(d * jax.lax.rsqrt(v + EPS) * g1 + b1, 0.0)
    f2_ref[...] = jnp.mean(y, axis=1)
    f = jnp.mean(nb2d.reshape(B, DEG, F), axis=1)
    x1 = (jnp.dot(x_ref[...].astype(jnp.bfloat16), w1xt,
                  preferred_element_type=jnp.float32)
          + jnp.dot(f.astype(jnp.bfloat16), w1nt_ref[...].astype(jnp.bfloat16),
                    preferred_element_type=jnp.float32))
    x1_ref[...] = x1
    s1_ref[...] = jnp.sum(x1.reshape(B // 8, 8, H), axis=0)
    ss1_ref[...] = jnp.sum((x1 * x1).reshape(B // 8, 8, H), axis=0)


def _k2(x1_ref, f2_ref, s1_ref, ss1_ref, w2xt_ref, w2nt_ref, g1_ref, b1_ref,
        x2_ref, s2_ref, ss2_ref):
    cnt = float(N * H)
    m1 = jnp.sum(s1_ref[...]) / cnt
    v1 = jnp.sum(ss1_ref[...]) / cnt - m1 * m1
    g1 = g1_ref[0, 0]
    b1 = b1_ref[0, 0]
    x1 = x1_ref[...]
    x1n = jnp.maximum((x1 - m1) * jax.lax.rsqrt(v1 + EPS) * g1 + b1, 0.0)
    x2 = (jnp.dot(x1n, w2xt_ref[...], preferred_element_type=jnp.float32)
          + jnp.dot(f2_ref[...], w2nt_ref[...],
                    preferred_element_type=jnp.float32))
    x2_ref[...] = x2
    s2_ref[...] = jnp.sum(x2.reshape(B // 8, 8, H), axis=0)
    ss2_ref[...] = jnp.sum((x2 * x2).reshape(B // 8, 8, H), axis=0)


def _k3(x2_ref, s2_ref, ss2_ref, wct_ref, bc_ref, g2_ref, b2_ref, out_ref):
    cnt = float(N * H)
    m2 = jnp.sum(s2_ref[...]) / cnt
    v2 = jnp.sum(ss2_ref[...]) / cnt - m2 * m2
    g2 = g2_ref[0, 0]
    b2 = b2_ref[0, 0]
    x2 = x2_ref[...]
    x2n = jnp.maximum((x2 - m2) * jax.lax.rsqrt(v2 + EPS) * g2 + b2, 0.0)
    out_ref[...] = (jnp.dot(x2n, wct_ref[...], preferred_element_type=jnp.float32)
                    + bc_ref[...])


def _smem11():
    return pl.BlockSpec(memory_space=pltpu.SMEM)


def _full():
    return pl.BlockSpec(memory_space=pltpu.VMEM)


@functools.partial(jax.jit)
def kernel(x, neighbor, W1x, W1n, W2x, W2n, g1, b1, g2, b2, Wc, bc):
    x2d = x.reshape(N, F)
    nb2d = neighbor.reshape(N * DEG, F)
    g1s = g1.reshape(1, 1)
    b1s = b1.reshape(1, 1)
    g2s = g2.reshape(1, 1)
    b2s = b2.reshape(1, 1)

    x1, f2, s1, ss1 = pl.pallas_call(
        _k1,
        grid=(NB,),
        in_specs=[
            pl.BlockSpec((B, F), lambda i: (i, 0)),
            pl.BlockSpec((B * DEG, F), lambda i: (i, 0)),
            _full(),
            _full(),
            _smem11(),
            _smem11(),
        ],
        out_specs=[
            pl.BlockSpec((B, H), lambda i: (i, 0)),
            pl.BlockSpec((B, H), lambda i: (i, 0)),
            pl.BlockSpec((8, H), lambda i: (i, 0)),
            pl.BlockSpec((8, H), lambda i: (i, 0)),
        ],
        out_shape=[
            jax.ShapeDtypeStruct((N, H), jnp.float32),
            jax.ShapeDtypeStruct((N, H), jnp.float32),
            jax.ShapeDtypeStruct((NB * 8, H), jnp.float32),
            jax.ShapeDtypeStruct((NB * 8, H), jnp.float32),
        ],
        compiler_params=pltpu.CompilerParams(
            dimension_semantics=("parallel",)),
    )(x2d, nb2d, W1x.T, W1n.T, g1s, b1s)

    x2, s2, ss2 = pl.pallas_call(
        _k2,
        grid=(NB,),
        in_specs=[
            pl.BlockSpec((B, H), lambda i: (i, 0)),
            pl.BlockSpec((B, H), lambda i: (i, 0)),
            pl.BlockSpec((NB * 8, H), lambda i: (0, 0)),
            pl.BlockSpec((NB * 8, H), lambda i: (0, 0)),
            _full(),
            _full(),
            _smem11(),
            _smem11(),
        ],
        out_specs=[
            pl.BlockSpec((B, H), lambda i: (i, 0)),
            pl.BlockSpec((8, H), lambda i: (i, 0)),
            pl.BlockSpec((8, H), lambda i: (i, 0)),
        ],
        out_shape=[
            jax.ShapeDtypeStruct((N, H), jnp.float32),
            jax.ShapeDtypeStruct((NB * 8, H), jnp.float32),
            jax.ShapeDtypeStruct((NB * 8, H), jnp.float32),
        ],
        compiler_params=pltpu.CompilerParams(
            dimension_semantics=("parallel",)),
    )(x1, f2, s1, ss1, W2x.T, W2n.T, g1s, b1s)

    out = pl.pallas_call(
        _k3,
        grid=(NB,),
        in_specs=[
            pl.BlockSpec((B, H), lambda i: (i, 0)),
            pl.BlockSpec((NB * 8, H), lambda i: (0, 0)),
            pl.BlockSpec((NB * 8, H), lambda i: (0, 0)),
            _full(),
            _full(),
            _smem11(),
            _smem11(),
        ],
        out_specs=pl.BlockSpec((B, C), lambda i: (i, 0)),
        out_shape=jax.ShapeDtypeStruct((N, C), jnp.float32),
        compiler_params=pltpu.CompilerParams(
            dimension_semantics=("parallel",)),
    )(x2, s2, ss2, Wc.T, bc.reshape(1, C), g2s, b2s)

    return out
